# baseline (speedup 1.0000x reference)
import functools

import jax
import jax.numpy as jnp
from jax import lax
from jax.experimental import pallas as pl
from jax.experimental.pallas import tpu as pltpu
from jax.experimental.pallas import tpu_sc as plsc

_N = 10000
_E = 320000
_D = 128
_NT = 4
_STEPS = 4

_NTILES = 32
_EPT = _E // _NTILES
_C = 100
_NCH = _EPT // _C
_NSEG = 4
_CPS = _NCH // _NSEG
_P = 2
_NP = 10240
_RPT = _NP // 16


def _gidx_body(ef_ref, src_ref, o_ref):
    best = ef_ref[0]
    idx = jnp.zeros_like(src_ref[...])
    for i in range(1, _NT):
        c = ef_ref[i]
        m = c > best
        idx = jnp.where(m, i, idx)
        best = jnp.where(m, c, best)
    o_ref[...] = idx * _N + src_ref[...]


def _gidx_call(ef_t, src2, interpret=False):
    return pl.pallas_call(
        _gidx_body,
        out_shape=jax.ShapeDtypeStruct((_E // _D, _D), jnp.int32),
        interpret=interpret,
    )(ef_t, src2)


def _y_body(h_ref, w_ref, b_ref, y_ref):
    y_ref[0] = (
        jnp.dot(h_ref[...], w_ref[0], preferred_element_type=jnp.float32)
        + b_ref[0]
    )


def _y_call(h, wt, b3, interpret=False):
    return pl.pallas_call(
        _y_body,
        grid=(_NT,),
        in_specs=[
            pl.BlockSpec((_N, _D), lambda i: (0, 0)),
            pl.BlockSpec((1, _D, _D), lambda i: (i, 0, 0)),
            pl.BlockSpec((1, 1, _D), lambda i: (i, 0, 0)),
        ],
        out_specs=pl.BlockSpec((1, _N, _D), lambda i: (i, 0, 0)),
        out_shape=jax.ShapeDtypeStruct((_NT, _N, _D), jnp.float32),
        interpret=interpret,
    )(h, wt, b3)


_BN = 2000


def _gru_body(p_ref, h_ref, wi_ref, wh_ref, bi_ref, bh_ref, o_ref):
    agg = p_ref[0] + p_ref[1]
    h = h_ref[...]
    gi = jnp.dot(agg, wi_ref[...], preferred_element_type=jnp.float32) + bi_ref[...]
    gh = jnp.dot(h, wh_ref[...], preferred_element_type=jnp.float32) + bh_ref[...]
    r = jax.nn.sigmoid(gi[:, :_D] + gh[:, :_D])
    z = jax.nn.sigmoid(gi[:, _D:2 * _D] + gh[:, _D:2 * _D])
    n = jnp.tanh(gi[:, 2 * _D:] + r * gh[:, 2 * _D:])
    o_ref[...] = (1.0 - z) * n + z * h


def _gru_call(parts, h, wiT, whT, bi2, bh2, interpret=False):
    return pl.pallas_call(
        _gru_body,
        grid=(_N // _BN,),
        in_specs=[
            pl.BlockSpec((2, _BN, _D), lambda i: (0, i, 0)),
            pl.BlockSpec((_BN, _D), lambda i: (i, 0)),
            pl.BlockSpec((_D, 3 * _D), lambda i: (0, 0)),
            pl.BlockSpec((_D, 3 * _D), lambda i: (0, 0)),
            pl.BlockSpec((1, 3 * _D), lambda i: (0, 0)),
            pl.BlockSpec((1, 3 * _D), lambda i: (0, 0)),
        ],
        out_specs=pl.BlockSpec((_BN, _D), lambda i: (i, 0)),
        out_shape=jax.ShapeDtypeStruct((_N, _D), jnp.float32),
        interpret=interpret,
    )(parts, h, wiT, whT, bi2, bh2)


def _gruy_body(p_ref, h_ref, wi_ref, wh_ref, bi_ref, bh_ref, wt_ref, be_ref,
               o_ref, y_ref):
    agg = p_ref[0] + p_ref[1]
    h = h_ref[...]
    gi = jnp.dot(agg, wi_ref[...], preferred_element_type=jnp.float32) + bi_ref[...]
    gh = jnp.dot(h, wh_ref[...], preferred_element_type=jnp.float32) + bh_ref[...]
    r = jax.nn.sigmoid(gi[:, :_D] + gh[:, :_D])
    z = jax.nn.sigmoid(gi[:, _D:2 * _D] + gh[:, _D:2 * _D])
    n = jnp.tanh(gi[:, 2 * _D:] + r * gh[:, 2 * _D:])
    hn = (1.0 - z) * n + z * h
    o_ref[...] = hn
    for t in range(_NT):
        y_ref[t] = (
            jnp.dot(hn, wt_ref[t], preferred_element_type=jnp.float32)
            + be_ref[t]
        )


def _gruy_call(parts, h, wiT, whT, bi2, bh2, wt, b3, interpret=False):
    return pl.pallas_call(
        _gruy_body,
        grid=(_N // _BN,),
        in_specs=[
            pl.BlockSpec((2, _BN, _D), lambda i: (0, i, 0)),
            pl.BlockSpec((_BN, _D), lambda i: (i, 0)),
            pl.BlockSpec((_D, 3 * _D), lambda i: (0, 0)),
            pl.BlockSpec((_D, 3 * _D), lambda i: (0, 0)),
            pl.BlockSpec((1, 3 * _D), lambda i: (0, 0)),
            pl.BlockSpec((1, 3 * _D), lambda i: (0, 0)),
            pl.BlockSpec((_NT, _D, _D), lambda i: (0, 0, 0)),
            pl.BlockSpec((_NT, 1, _D), lambda i: (0, 0, 0)),
        ],
        out_specs=[
            pl.BlockSpec((_BN, _D), lambda i: (i, 0)),
            pl.BlockSpec((_NT, _BN, _D), lambda i: (0, i, 0)),
        ],
        out_shape=[
            jax.ShapeDtypeStruct((_N, _D), jnp.float32),
            jax.ShapeDtypeStruct((_NT, _N, _D), jnp.float32),
        ],
        interpret=interpret,
    )(parts, h, wiT, whT, bi2, bh2, wt, b3)


@functools.lru_cache(maxsize=1)
def _make_sc_aggregate():
    mesh = plsc.VectorSubcoreMesh(core_axis_name="c", subcore_axis_name="s")

    @functools.partial(
        pl.kernel,
        out_type=jax.ShapeDtypeStruct((2, _NP, _D), jnp.float32),
        mesh=mesh,
        scratch_types=(
            [pltpu.VMEM((_CPS, _C), jnp.int32),
             pltpu.VMEM((_CPS, _C), jnp.int32)]
            + [pltpu.VMEM((_C, _D), jnp.float32) for _ in range(_P)]
            + [pltpu.VMEM_SHARED((_NP, _D), jnp.float32)]
            + [pltpu.SemaphoreType.DMA for _ in range(2 * _P + 1)]
        ),
    )
    def _sc_aggregate(y_hbm, gidx_hbm, dst_hbm, zero_hbm, out_hbm,
                      gidx_v, dst_v, *rest):
        bufs = rest[:_P]
        acc_sh = rest[_P]
        gsems = rest[_P + 1:2 * _P + 1]
        ssems = rest[2 * _P + 1:3 * _P + 1]
        zsem = rest[3 * _P + 1]
        cid = lax.axis_index("c")
        sid = lax.axis_index("s")
        tid = cid * 16 + sid

        zdma = pltpu.async_copy(
            zero_hbm, acc_sh.at[pl.ds(sid * _RPT, _RPT)], zsem)
        slab_g = pltpu.async_copy(gidx_hbm.at[tid, 0], gidx_v, gsems[0])
        slab_d = pltpu.async_copy(dst_hbm.at[tid, 0], dst_v, gsems[1])
        slab_g.wait()
        slab_d.wait()
        zdma.wait()
        plsc.subcore_barrier()

        def seg_body(seg, carry):
            def gather(j):
                p = j % _P
                return pltpu.async_copy(
                    y_hbm.at[gidx_v.at[j]], bufs[p], gsems[p])

            copies = [None] * _CPS
            scats = [None] * _CPS
            for j in range(min(_P, _CPS)):
                copies[j] = gather(j)
            for j in range(_CPS):
                p = j % _P
                copies[j].wait()
                scats[j] = pltpu.async_copy(
                    bufs[p], acc_sh.at[dst_v.at[j]], ssems[p], add=True)
                k = j + _P - 1
                if _P <= k < _CPS:
                    scats[k - _P].wait()
                    copies[k] = gather(k)
            for j in range(_CPS - _P, _CPS):
                scats[j].wait()

            @pl.when(seg < _NSEG - 1)
            def _next_slabs():
                pltpu.async_copy(gidx_hbm.at[tid, seg + 1], gidx_v,
                                 gsems[0]).wait()
                pltpu.async_copy(dst_hbm.at[tid, seg + 1], dst_v,
                                 gsems[1]).wait()

            return carry

        lax.fori_loop(0, _NSEG, seg_body, 0)
        plsc.subcore_barrier()

        pltpu.sync_copy(acc_sh.at[pl.ds(sid * _RPT, _RPT)],
                        out_hbm.at[cid, pl.ds(sid * _RPT, _RPT)])

    return _sc_aggregate


def kernel(node_feat, edge_index, edge_feat, W_edge, b_edge,
           W_ih, W_hh, b_ih, b_hh):
    src = edge_index[0].astype(jnp.int32)
    dst = edge_index[1].astype(jnp.int32)

    ef_t = jnp.transpose(edge_feat).reshape(_NT, _E // _D, _D)
    src2 = src.reshape(_E // _D, _D)
    gidx4 = _gidx_call(ef_t, src2).reshape(_NTILES, _NSEG, _CPS, _C)
    dst4 = dst.reshape(_NTILES, _NSEG, _CPS, _C)

    wt = jnp.transpose(W_edge, (0, 2, 1))
    b3 = b_edge.reshape(_NT, 1, _D)
    wiT = jnp.transpose(W_ih)
    whT = jnp.transpose(W_hh)
    bi2 = b_ih.reshape(1, 3 * _D)
    bh2 = b_hh.reshape(1, 3 * _D)
    zeros = jnp.zeros((_RPT, _D), jnp.float32)

    h = node_feat
    y = _y_call(h, wt, b3).reshape(_NT * _N, _D)
    for step in range(_STEPS):
        parts = _make_sc_aggregate()(y, gidx4, dst4, zeros)
        if step < _STEPS - 1:
            h, y4 = _gruy_call(parts, h, wiT, whT, bi2, bh2, wt, b3)
            y = y4.reshape(_NT * _N, _D)
        else:
            h = _gru_call(parts, h, wiT, whT, bi2, bh2)
    return h


# speedup vs baseline: 1.2221x; 1.2221x over previous
import functools

import jax
import jax.numpy as jnp
from jax import lax
from jax.experimental import pallas as pl
from jax.experimental.pallas import tpu as pltpu
from jax.experimental.pallas import tpu_sc as plsc

_N = 10000
_E = 320000
_D = 128
_NT = 4
_STEPS = 4

_NTILES = 32
_EPT = _E // _NTILES
_C = 80
_NCH = _EPT // _C
_NSEG = 5
_CPS = _NCH // _NSEG
_P = 3
_NP = 10240
_RPT = _NP // 16


def _gidx_body(ef_ref, src_ref, o_ref):
    best = ef_ref[0]
    idx = jnp.zeros_like(src_ref[...])
    for i in range(1, _NT):
        c = ef_ref[i]
        m = c > best
        idx = jnp.where(m, i, idx)
        best = jnp.where(m, c, best)
    o_ref[...] = idx * _N + src_ref[...]


def _gidx_call(ef_t, src2, interpret=False):
    return pl.pallas_call(
        _gidx_body,
        out_shape=jax.ShapeDtypeStruct((_E // _D, _D), jnp.int32),
        interpret=interpret,
    )(ef_t, src2)


def _y_body(h_ref, w_ref, b_ref, y_ref):
    y_ref[0] = (
        jnp.dot(h_ref[...].astype(jnp.bfloat16),
                w_ref[0].astype(jnp.bfloat16),
                preferred_element_type=jnp.float32)
        + b_ref[0]
    )


def _y_call(h, wt, b3, interpret=False):
    return pl.pallas_call(
        _y_body,
        grid=(_NT,),
        in_specs=[
            pl.BlockSpec((_N, _D), lambda i: (0, 0)),
            pl.BlockSpec((1, _D, _D), lambda i: (i, 0, 0)),
            pl.BlockSpec((1, 1, _D), lambda i: (i, 0, 0)),
        ],
        out_specs=pl.BlockSpec((1, _N, _D), lambda i: (i, 0, 0)),
        out_shape=jax.ShapeDtypeStruct((_NT, _N, _D), jnp.float32),
        interpret=interpret,
    )(h, wt, b3)


_BN = 2000


def _gru_body(p_ref, h_ref, wi_ref, wh_ref, bi_ref, bh_ref, o_ref):
    agg = p_ref[0] + p_ref[1]
    h = h_ref[...]
    gi = jnp.dot(agg.astype(jnp.bfloat16),
                 wi_ref[...].astype(jnp.bfloat16),
                 preferred_element_type=jnp.float32) + bi_ref[...]
    gh = jnp.dot(h.astype(jnp.bfloat16),
                 wh_ref[...].astype(jnp.bfloat16),
                 preferred_element_type=jnp.float32) + bh_ref[...]
    r = jax.nn.sigmoid(gi[:, :_D] + gh[:, :_D])
    z = jax.nn.sigmoid(gi[:, _D:2 * _D] + gh[:, _D:2 * _D])
    n = jnp.tanh(gi[:, 2 * _D:] + r * gh[:, 2 * _D:])
    o_ref[...] = (1.0 - z) * n + z * h


def _gru_call(parts, h, wiT, whT, bi2, bh2, interpret=False):
    return pl.pallas_call(
        _gru_body,
        grid=(_N // _BN,),
        in_specs=[
            pl.BlockSpec((2, _BN, _D), lambda i: (0, i, 0)),
            pl.BlockSpec((_BN, _D), lambda i: (i, 0)),
            pl.BlockSpec((_D, 3 * _D), lambda i: (0, 0)),
            pl.BlockSpec((_D, 3 * _D), lambda i: (0, 0)),
            pl.BlockSpec((1, 3 * _D), lambda i: (0, 0)),
            pl.BlockSpec((1, 3 * _D), lambda i: (0, 0)),
        ],
        out_specs=pl.BlockSpec((_BN, _D), lambda i: (i, 0)),
        out_shape=jax.ShapeDtypeStruct((_N, _D), jnp.float32),
        interpret=interpret,
    )(parts, h, wiT, whT, bi2, bh2)


def _gruy_body(p_ref, h_ref, wi_ref, wh_ref, bi_ref, bh_ref, wt_ref, be_ref,
               o_ref, y_ref):
    agg = p_ref[0] + p_ref[1]
    h = h_ref[...]
    gi = jnp.dot(agg.astype(jnp.bfloat16),
                 wi_ref[...].astype(jnp.bfloat16),
                 preferred_element_type=jnp.float32) + bi_ref[...]
    gh = jnp.dot(h.astype(jnp.bfloat16),
                 wh_ref[...].astype(jnp.bfloat16),
                 preferred_element_type=jnp.float32) + bh_ref[...]
    r = jax.nn.sigmoid(gi[:, :_D] + gh[:, :_D])
    z = jax.nn.sigmoid(gi[:, _D:2 * _D] + gh[:, _D:2 * _D])
    n = jnp.tanh(gi[:, 2 * _D:] + r * gh[:, 2 * _D:])
    hn = (1.0 - z) * n + z * h
    o_ref[...] = hn
    for t in range(_NT):
        y_ref[t] = (
            jnp.dot(hn.astype(jnp.bfloat16),
                    wt_ref[t].astype(jnp.bfloat16),
                    preferred_element_type=jnp.float32)
            + be_ref[t]
        )


def _gruy_call(parts, h, wiT, whT, bi2, bh2, wt, b3, interpret=False):
    return pl.pallas_call(
        _gruy_body,
        grid=(_N // _BN,),
        in_specs=[
            pl.BlockSpec((2, _BN, _D), lambda i: (0, i, 0)),
            pl.BlockSpec((_BN, _D), lambda i: (i, 0)),
            pl.BlockSpec((_D, 3 * _D), lambda i: (0, 0)),
            pl.BlockSpec((_D, 3 * _D), lambda i: (0, 0)),
            pl.BlockSpec((1, 3 * _D), lambda i: (0, 0)),
            pl.BlockSpec((1, 3 * _D), lambda i: (0, 0)),
            pl.BlockSpec((_NT, _D, _D), lambda i: (0, 0, 0)),
            pl.BlockSpec((_NT, 1, _D), lambda i: (0, 0, 0)),
        ],
        out_specs=[
            pl.BlockSpec((_BN, _D), lambda i: (i, 0)),
            pl.BlockSpec((_NT, _BN, _D), lambda i: (0, i, 0)),
        ],
        out_shape=[
            jax.ShapeDtypeStruct((_N, _D), jnp.float32),
            jax.ShapeDtypeStruct((_NT, _N, _D), jnp.float32),
        ],
        interpret=interpret,
    )(parts, h, wiT, whT, bi2, bh2, wt, b3)


@functools.lru_cache(maxsize=1)
def _make_sc_aggregate():
    mesh = plsc.VectorSubcoreMesh(core_axis_name="c", subcore_axis_name="s")

    @functools.partial(
        pl.kernel,
        out_type=jax.ShapeDtypeStruct((2, _NP, _D), jnp.float32),
        mesh=mesh,
        scratch_types=(
            [pltpu.VMEM((_CPS, _C), jnp.int32),
             pltpu.VMEM((_CPS, _C), jnp.int32)]
            + [pltpu.VMEM((_C, _D), jnp.float32) for _ in range(_P)]
            + [pltpu.VMEM_SHARED((_NP, _D), jnp.float32)]
            + [pltpu.SemaphoreType.DMA for _ in range(2 * _P + 1)]
        ),
    )
    def _sc_aggregate(y_hbm, gidx_hbm, dst_hbm, zero_hbm, out_hbm,
                      gidx_v, dst_v, *rest):
        bufs = rest[:_P]
        acc_sh = rest[_P]
        gsems = rest[_P + 1:2 * _P + 1]
        ssems = rest[2 * _P + 1:3 * _P + 1]
        zsem = rest[3 * _P + 1]
        cid = lax.axis_index("c")
        sid = lax.axis_index("s")
        tid = cid * 16 + sid

        zdma = pltpu.async_copy(
            zero_hbm, acc_sh.at[pl.ds(sid * _RPT, _RPT)], zsem)
        slab_g = pltpu.async_copy(gidx_hbm.at[tid, 0], gidx_v, gsems[0])
        slab_d = pltpu.async_copy(dst_hbm.at[tid, 0], dst_v, gsems[1])
        slab_g.wait()
        slab_d.wait()
        zdma.wait()
        plsc.subcore_barrier()

        def seg_body(seg, carry):
            def gather(j):
                p = j % _P
                return pltpu.async_copy(
                    y_hbm.at[gidx_v.at[j]], bufs[p], gsems[p])

            copies = [None] * _CPS
            scats = [None] * _CPS
            for j in range(min(_P, _CPS)):
                copies[j] = gather(j)
            for j in range(_CPS):
                p = j % _P
                copies[j].wait()
                scats[j] = pltpu.async_copy(
                    bufs[p], acc_sh.at[dst_v.at[j]], ssems[p], add=True)
                k = j + _P - 1
                if _P <= k < _CPS:
                    scats[k - _P].wait()
                    copies[k] = gather(k)
            for j in range(_CPS - _P, _CPS):
                scats[j].wait()

            @pl.when(seg < _NSEG - 1)
            def _next_slabs():
                pltpu.async_copy(gidx_hbm.at[tid, seg + 1], gidx_v,
                                 gsems[0]).wait()
                pltpu.async_copy(dst_hbm.at[tid, seg + 1], dst_v,
                                 gsems[1]).wait()

            return carry

        lax.fori_loop(0, _NSEG, seg_body, 0)
        plsc.subcore_barrier()

        pltpu.sync_copy(acc_sh.at[pl.ds(sid * _RPT, _RPT)],
                        out_hbm.at[cid, pl.ds(sid * _RPT, _RPT)])

    return _sc_aggregate


def kernel(node_feat, edge_index, edge_feat, W_edge, b_edge,
           W_ih, W_hh, b_ih, b_hh):
    src = edge_index[0].astype(jnp.int32)
    dst = edge_index[1].astype(jnp.int32)

    ef_t = jnp.transpose(edge_feat).reshape(_NT, _E // _D, _D)
    src2 = src.reshape(_E // _D, _D)
    gidx4 = _gidx_call(ef_t, src2).reshape(_NTILES, _NSEG, _CPS, _C)
    dst4 = dst.reshape(_NTILES, _NSEG, _CPS, _C)

    wt = jnp.transpose(W_edge, (0, 2, 1))
    b3 = b_edge.reshape(_NT, 1, _D)
    wiT = jnp.transpose(W_ih)
    whT = jnp.transpose(W_hh)
    bi2 = b_ih.reshape(1, 3 * _D)
    bh2 = b_hh.reshape(1, 3 * _D)
    zeros = jnp.zeros((_RPT, _D), jnp.float32)

    h = node_feat
    y = _y_call(h, wt, b3).reshape(_NT * _N, _D)
    for step in range(_STEPS):
        parts = _make_sc_aggregate()(y, gidx4, dst4, zeros)
        if step < _STEPS - 1:
            h, y4 = _gruy_call(parts, h, wiT, whT, bi2, bh2, wt, b3)
            y = y4.reshape(_NT * _N, _D)
        else:
            h = _gru_call(parts, h, wiT, whT, bi2, bh2)
    return h


# speedup vs baseline: 1.2310x; 1.0073x over previous
import functools

import jax
import jax.numpy as jnp
from jax import lax
from jax.experimental import pallas as pl
from jax.experimental.pallas import tpu as pltpu
from jax.experimental.pallas import tpu_sc as plsc

_N = 10000
_E = 320000
_D = 128
_NT = 4
_STEPS = 4

_NTILES = 32
_EPT = _E // _NTILES
_C = 80
_NCH = _EPT // _C
_NSEG = 5
_CPS = _NCH // _NSEG
_P = 3
_NP = 10240
_RPT = _NP // 16


def _gidx_body(ef_ref, src_ref, o_ref):
    best = ef_ref[0]
    idx = jnp.zeros_like(src_ref[...])
    for i in range(1, _NT):
        c = ef_ref[i]
        m = c > best
        idx = jnp.where(m, i, idx)
        best = jnp.where(m, c, best)
    o_ref[...] = idx * _N + src_ref[...]


def _gidx_call(ef_t, src2, interpret=False):
    return pl.pallas_call(
        _gidx_body,
        out_shape=jax.ShapeDtypeStruct((_E // _D, _D), jnp.int32),
        interpret=interpret,
    )(ef_t, src2)


def _y_body(h_ref, w_ref, b_ref, y_ref):
    y_ref[0] = (
        jnp.dot(h_ref[...], w_ref[0], preferred_element_type=jnp.float32)
        + b_ref[0]
    )


def _y_call(h, wt, b3, interpret=False):
    return pl.pallas_call(
        _y_body,
        grid=(_NT,),
        in_specs=[
            pl.BlockSpec((_N, _D), lambda i: (0, 0)),
            pl.BlockSpec((1, _D, _D), lambda i: (i, 0, 0)),
            pl.BlockSpec((1, 1, _D), lambda i: (i, 0, 0)),
        ],
        out_specs=pl.BlockSpec((1, _N, _D), lambda i: (i, 0, 0)),
        out_shape=jax.ShapeDtypeStruct((_NT, _N, _D), jnp.float32),
        interpret=interpret,
    )(h, wt, b3)


def _y0_body(h_ref, w_ref, b_ref, ef_ref, src_ref, y_ref, g_ref):
    y_ref[0] = (
        jnp.dot(h_ref[...], w_ref[0], preferred_element_type=jnp.float32)
        + b_ref[0]
    )

    @pl.when(pl.program_id(0) == 0)
    def _gidx():
        best = ef_ref[0]
        idx = jnp.zeros_like(src_ref[...])
        for i in range(1, _NT):
            c = ef_ref[i]
            m = c > best
            idx = jnp.where(m, i, idx)
            best = jnp.where(m, c, best)
        g_ref[...] = idx * _N + src_ref[...]


def _y0_call(h, wt, b3, ef_t, src2, interpret=False):
    return pl.pallas_call(
        _y0_body,
        grid=(_NT,),
        in_specs=[
            pl.BlockSpec((_N, _D), lambda i: (0, 0)),
            pl.BlockSpec((1, _D, _D), lambda i: (i, 0, 0)),
            pl.BlockSpec((1, 1, _D), lambda i: (i, 0, 0)),
            pl.BlockSpec((_NT, _E // _D, _D), lambda i: (0, 0, 0)),
            pl.BlockSpec((_E // _D, _D), lambda i: (0, 0)),
        ],
        out_specs=[
            pl.BlockSpec((1, _N, _D), lambda i: (i, 0, 0)),
            pl.BlockSpec((_E // _D, _D), lambda i: (0, 0)),
        ],
        out_shape=[
            jax.ShapeDtypeStruct((_NT, _N, _D), jnp.float32),
            jax.ShapeDtypeStruct((_E // _D, _D), jnp.int32),
        ],
        interpret=interpret,
    )(h, wt, b3, ef_t, src2)


_BN = 2000


def _gru_body(p_ref, h_ref, wi_ref, wh_ref, bi_ref, bh_ref, o_ref):
    agg = p_ref[0] + p_ref[1]
    h = h_ref[...]
    gi = jnp.dot(agg, wi_ref[...], preferred_element_type=jnp.float32) + bi_ref[...]
    gh = jnp.dot(h, wh_ref[...], preferred_element_type=jnp.float32) + bh_ref[...]
    r = jax.nn.sigmoid(gi[:, :_D] + gh[:, :_D])
    z = jax.nn.sigmoid(gi[:, _D:2 * _D] + gh[:, _D:2 * _D])
    n = jnp.tanh(gi[:, 2 * _D:] + r * gh[:, 2 * _D:])
    o_ref[...] = (1.0 - z) * n + z * h


def _gru_call(parts, h, wiT, whT, bi2, bh2, interpret=False):
    return pl.pallas_call(
        _gru_body,
        grid=(_N // _BN,),
        in_specs=[
            pl.BlockSpec((2, _BN, _D), lambda i: (0, i, 0)),
            pl.BlockSpec((_BN, _D), lambda i: (i, 0)),
            pl.BlockSpec((_D, 3 * _D), lambda i: (0, 0)),
            pl.BlockSpec((_D, 3 * _D), lambda i: (0, 0)),
            pl.BlockSpec((1, 3 * _D), lambda i: (0, 0)),
            pl.BlockSpec((1, 3 * _D), lambda i: (0, 0)),
        ],
        out_specs=pl.BlockSpec((_BN, _D), lambda i: (i, 0)),
        out_shape=jax.ShapeDtypeStruct((_N, _D), jnp.float32),
        interpret=interpret,
    )(parts, h, wiT, whT, bi2, bh2)


def _gruy_body(p_ref, h_ref, wi_ref, wh_ref, bi_ref, bh_ref, wt_ref, be_ref,
               o_ref, y_ref):
    agg = p_ref[0] + p_ref[1]
    h = h_ref[...]
    gi = jnp.dot(agg, wi_ref[...], preferred_element_type=jnp.float32) + bi_ref[...]
    gh = jnp.dot(h, wh_ref[...], preferred_element_type=jnp.float32) + bh_ref[...]
    r = jax.nn.sigmoid(gi[:, :_D] + gh[:, :_D])
    z = jax.nn.sigmoid(gi[:, _D:2 * _D] + gh[:, _D:2 * _D])
    n = jnp.tanh(gi[:, 2 * _D:] + r * gh[:, 2 * _D:])
    hn = (1.0 - z) * n + z * h
    o_ref[...] = hn
    for t in range(_NT):
        y_ref[t] = (
            jnp.dot(hn, wt_ref[t], preferred_element_type=jnp.float32)
            + be_ref[t]
        )


def _gruy_call(parts, h, wiT, whT, bi2, bh2, wt, b3, interpret=False):
    return pl.pallas_call(
        _gruy_body,
        grid=(_N // _BN,),
        in_specs=[
            pl.BlockSpec((2, _BN, _D), lambda i: (0, i, 0)),
            pl.BlockSpec((_BN, _D), lambda i: (i, 0)),
            pl.BlockSpec((_D, 3 * _D), lambda i: (0, 0)),
            pl.BlockSpec((_D, 3 * _D), lambda i: (0, 0)),
            pl.BlockSpec((1, 3 * _D), lambda i: (0, 0)),
            pl.BlockSpec((1, 3 * _D), lambda i: (0, 0)),
            pl.BlockSpec((_NT, _D, _D), lambda i: (0, 0, 0)),
            pl.BlockSpec((_NT, 1, _D), lambda i: (0, 0, 0)),
        ],
        out_specs=[
            pl.BlockSpec((_BN, _D), lambda i: (i, 0)),
            pl.BlockSpec((_NT, _BN, _D), lambda i: (0, i, 0)),
        ],
        out_shape=[
            jax.ShapeDtypeStruct((_N, _D), jnp.float32),
            jax.ShapeDtypeStruct((_NT, _N, _D), jnp.float32),
        ],
        interpret=interpret,
    )(parts, h, wiT, whT, bi2, bh2, wt, b3)


@functools.lru_cache(maxsize=1)
def _make_sc_aggregate():
    mesh = plsc.VectorSubcoreMesh(core_axis_name="c", subcore_axis_name="s")

    @functools.partial(
        pl.kernel,
        out_type=jax.ShapeDtypeStruct((2, _NP, _D), jnp.float32),
        mesh=mesh,
        scratch_types=(
            [pltpu.VMEM((_CPS, _C), jnp.int32),
             pltpu.VMEM((_CPS, _C), jnp.int32)]
            + [pltpu.VMEM((_C, _D), jnp.float32) for _ in range(_P)]
            + [pltpu.VMEM_SHARED((_NP, _D), jnp.float32)]
            + [pltpu.SemaphoreType.DMA for _ in range(2 * _P + 1)]
        ),
    )
    def _sc_aggregate(y_hbm, gidx_hbm, dst_hbm, zero_hbm, out_hbm,
                      gidx_v, dst_v, *rest):
        bufs = rest[:_P]
        acc_sh = rest[_P]
        gsems = rest[_P + 1:2 * _P + 1]
        ssems = rest[2 * _P + 1:3 * _P + 1]
        zsem = rest[3 * _P + 1]
        cid = lax.axis_index("c")
        sid = lax.axis_index("s")
        tid = cid * 16 + sid

        zdma = pltpu.async_copy(
            zero_hbm, acc_sh.at[pl.ds(sid * _RPT, _RPT)], zsem)
        slab_g = pltpu.async_copy(gidx_hbm.at[tid, 0], gidx_v, gsems[0])
        slab_d = pltpu.async_copy(dst_hbm.at[tid, 0], dst_v, gsems[1])
        slab_g.wait()
        slab_d.wait()
        zdma.wait()
        plsc.subcore_barrier()

        def seg_body(seg, carry):
            def gather(j):
                p = j % _P
                return pltpu.async_copy(
                    y_hbm.at[gidx_v.at[j]], bufs[p], gsems[p])

            copies = [None] * _CPS
            scats = [None] * _CPS
            for j in range(min(_P, _CPS)):
                copies[j] = gather(j)
            for j in range(_CPS):
                p = j % _P
                copies[j].wait()
                scats[j] = pltpu.async_copy(
                    bufs[p], acc_sh.at[dst_v.at[j]], ssems[p], add=True)
                k = j + _P - 1
                if _P <= k < _CPS:
                    scats[k - _P].wait()
                    copies[k] = gather(k)
            for j in range(_CPS - _P, _CPS):
                scats[j].wait()

            @pl.when(seg < _NSEG - 1)
            def _next_slabs():
                pltpu.async_copy(gidx_hbm.at[tid, seg + 1], gidx_v,
                                 gsems[0]).wait()
                pltpu.async_copy(dst_hbm.at[tid, seg + 1], dst_v,
                                 gsems[1]).wait()

            return carry

        lax.fori_loop(0, _NSEG, seg_body, 0)
        plsc.subcore_barrier()

        pltpu.sync_copy(acc_sh.at[pl.ds(sid * _RPT, _RPT)],
                        out_hbm.at[cid, pl.ds(sid * _RPT, _RPT)])

    return _sc_aggregate


def kernel(node_feat, edge_index, edge_feat, W_edge, b_edge,
           W_ih, W_hh, b_ih, b_hh):
    src = edge_index[0].astype(jnp.int32)
    dst = edge_index[1].astype(jnp.int32)

    ef_t = jnp.transpose(edge_feat).reshape(_NT, _E // _D, _D)
    src2 = src.reshape(_E // _D, _D)
    dst4 = dst.reshape(_NTILES, _NSEG, _CPS, _C)

    wt = jnp.transpose(W_edge, (0, 2, 1))
    b3 = b_edge.reshape(_NT, 1, _D)
    wiT = jnp.transpose(W_ih)
    whT = jnp.transpose(W_hh)
    bi2 = b_ih.reshape(1, 3 * _D)
    bh2 = b_hh.reshape(1, 3 * _D)
    zeros = jnp.zeros((_RPT, _D), jnp.float32)

    h = node_feat
    y4, gidx2 = _y0_call(h, wt, b3, ef_t, src2)
    y = y4.reshape(_NT * _N, _D)
    gidx4 = gidx2.reshape(_NTILES, _NSEG, _CPS, _C)
    for step in range(_STEPS):
        parts = _make_sc_aggregate()(y, gidx4, dst4, zeros)
        if step < _STEPS - 1:
            h, y4 = _gruy_call(parts, h, wiT, whT, bi2, bh2, wt, b3)
            y = y4.reshape(_NT * _N, _D)
        else:
            h = _gru_call(parts, h, wiT, whT, bi2, bh2)
    return h


# speedup vs baseline: 1.2545x; 1.0191x over previous
import functools

import jax
import jax.numpy as jnp
from jax import lax
from jax.experimental import pallas as pl
from jax.experimental.pallas import tpu as pltpu
from jax.experimental.pallas import tpu_sc as plsc

_N = 10000
_E = 320000
_D = 128
_NT = 4
_STEPS = 4

_NTILES = 32
_EPT = _E // _NTILES
_C = 80
_NCH = _EPT // _C
_NSEG = 5
_CPS = _NCH // _NSEG
_P = 3
_NP = 10240
_RPT = _NP // 16


def _gidx_body(ef_ref, src_ref, o_ref):
    best = ef_ref[0]
    idx = jnp.zeros_like(src_ref[...])
    for i in range(1, _NT):
        c = ef_ref[i]
        m = c > best
        idx = jnp.where(m, i, idx)
        best = jnp.where(m, c, best)
    o_ref[...] = idx * _N + src_ref[...]


def _gidx_call(ef_t, src2, interpret=False):
    return pl.pallas_call(
        _gidx_body,
        out_shape=jax.ShapeDtypeStruct((_E // _D, _D), jnp.int32),
        interpret=interpret,
    )(ef_t, src2)


def _y_body(h_ref, w_ref, b_ref, y_ref):
    y_ref[0] = (
        jnp.dot(h_ref[...], w_ref[0], preferred_element_type=jnp.float32)
        + b_ref[0]
    )


def _y_call(h, wt, b3, interpret=False):
    return pl.pallas_call(
        _y_body,
        grid=(_NT,),
        in_specs=[
            pl.BlockSpec((_N, _D), lambda i: (0, 0)),
            pl.BlockSpec((1, _D, _D), lambda i: (i, 0, 0)),
            pl.BlockSpec((1, 1, _D), lambda i: (i, 0, 0)),
        ],
        out_specs=pl.BlockSpec((1, _N, _D), lambda i: (i, 0, 0)),
        out_shape=jax.ShapeDtypeStruct((_NT, _N, _D), jnp.float32),
        interpret=interpret,
    )(h, wt, b3)


def _y0_body(h_ref, w_ref, b_ref, ef_ref, src_ref, y_ref, g_ref):
    y_ref[0] = (
        jnp.dot(h_ref[...], w_ref[0], preferred_element_type=jnp.float32)
        + b_ref[0]
    )

    @pl.when(pl.program_id(0) == 0)
    def _gidx():
        best = ef_ref[0]
        idx = jnp.zeros_like(src_ref[...])
        for i in range(1, _NT):
            c = ef_ref[i]
            m = c > best
            idx = jnp.where(m, i, idx)
            best = jnp.where(m, c, best)
        g_ref[...] = idx * _N + src_ref[...]


def _y0_call(h, wt, b3, ef_t, src2, interpret=False):
    return pl.pallas_call(
        _y0_body,
        grid=(_NT,),
        in_specs=[
            pl.BlockSpec((_N, _D), lambda i: (0, 0)),
            pl.BlockSpec((1, _D, _D), lambda i: (i, 0, 0)),
            pl.BlockSpec((1, 1, _D), lambda i: (i, 0, 0)),
            pl.BlockSpec((_NT, _E // _D, _D), lambda i: (0, 0, 0)),
            pl.BlockSpec((_E // _D, _D), lambda i: (0, 0)),
        ],
        out_specs=[
            pl.BlockSpec((1, _N, _D), lambda i: (i, 0, 0)),
            pl.BlockSpec((_E // _D, _D), lambda i: (0, 0)),
        ],
        out_shape=[
            jax.ShapeDtypeStruct((_NT, _N, _D), jnp.float32),
            jax.ShapeDtypeStruct((_E // _D, _D), jnp.int32),
        ],
        interpret=interpret,
    )(h, wt, b3, ef_t, src2)


_BN = 2000


def _gru_body(p_ref, h_ref, wi_ref, wh_ref, bi_ref, bh_ref, o_ref):
    agg = p_ref[0] + p_ref[1]
    h = h_ref[...]
    gi = jnp.dot(agg, wi_ref[...], preferred_element_type=jnp.float32) + bi_ref[...]
    gh = jnp.dot(h, wh_ref[...], preferred_element_type=jnp.float32) + bh_ref[...]
    r = jax.nn.sigmoid(gi[:, :_D] + gh[:, :_D])
    z = jax.nn.sigmoid(gi[:, _D:2 * _D] + gh[:, _D:2 * _D])
    n = jnp.tanh(gi[:, 2 * _D:] + r * gh[:, 2 * _D:])
    o_ref[...] = (1.0 - z) * n + z * h


def _gru_call(parts, h, wiT, whT, bi2, bh2, interpret=False):
    return pl.pallas_call(
        _gru_body,
        grid=(_N // _BN,),
        in_specs=[
            pl.BlockSpec((2, _BN, _D), lambda i: (0, i, 0)),
            pl.BlockSpec((_BN, _D), lambda i: (i, 0)),
            pl.BlockSpec((_D, 3 * _D), lambda i: (0, 0)),
            pl.BlockSpec((_D, 3 * _D), lambda i: (0, 0)),
            pl.BlockSpec((1, 3 * _D), lambda i: (0, 0)),
            pl.BlockSpec((1, 3 * _D), lambda i: (0, 0)),
        ],
        out_specs=pl.BlockSpec((_BN, _D), lambda i: (i, 0)),
        out_shape=jax.ShapeDtypeStruct((_N, _D), jnp.float32),
        interpret=interpret,
    )(parts, h, wiT, whT, bi2, bh2)


def _gruy_body(p_ref, h_ref, wi_ref, wh_ref, bi_ref, bh_ref, wt_ref, be_ref,
               o_ref, y_ref):
    agg = p_ref[0] + p_ref[1]
    h = h_ref[...]
    gi = jnp.dot(agg, wi_ref[...], preferred_element_type=jnp.float32) + bi_ref[...]
    gh = jnp.dot(h, wh_ref[...], preferred_element_type=jnp.float32) + bh_ref[...]
    r = jax.nn.sigmoid(gi[:, :_D] + gh[:, :_D])
    z = jax.nn.sigmoid(gi[:, _D:2 * _D] + gh[:, _D:2 * _D])
    n = jnp.tanh(gi[:, 2 * _D:] + r * gh[:, 2 * _D:])
    hn = (1.0 - z) * n + z * h
    o_ref[...] = hn
    for t in range(_NT):
        y_ref[t] = (
            jnp.dot(hn, wt_ref[t], preferred_element_type=jnp.float32)
            + be_ref[t]
        )


def _gruy_call(parts, h, wiT, whT, bi2, bh2, wt, b3, interpret=False):
    return pl.pallas_call(
        _gruy_body,
        grid=(_N // _BN,),
        in_specs=[
            pl.BlockSpec((2, _BN, _D), lambda i: (0, i, 0)),
            pl.BlockSpec((_BN, _D), lambda i: (i, 0)),
            pl.BlockSpec((_D, 3 * _D), lambda i: (0, 0)),
            pl.BlockSpec((_D, 3 * _D), lambda i: (0, 0)),
            pl.BlockSpec((1, 3 * _D), lambda i: (0, 0)),
            pl.BlockSpec((1, 3 * _D), lambda i: (0, 0)),
            pl.BlockSpec((_NT, _D, _D), lambda i: (0, 0, 0)),
            pl.BlockSpec((_NT, 1, _D), lambda i: (0, 0, 0)),
        ],
        out_specs=[
            pl.BlockSpec((_BN, _D), lambda i: (i, 0)),
            pl.BlockSpec((_NT, _BN, _D), lambda i: (0, i, 0)),
        ],
        out_shape=[
            jax.ShapeDtypeStruct((_N, _D), jnp.float32),
            jax.ShapeDtypeStruct((_NT, _N, _D), jnp.float32),
        ],
        interpret=interpret,
    )(parts, h, wiT, whT, bi2, bh2, wt, b3)


@functools.lru_cache(maxsize=1)
def _make_sc_aggregate():
    mesh = plsc.VectorSubcoreMesh(core_axis_name="c", subcore_axis_name="s")

    @functools.partial(
        pl.kernel,
        out_type=jax.ShapeDtypeStruct((2, _NP, _D), jnp.float32),
        mesh=mesh,
        scratch_types=(
            [pltpu.VMEM((_CPS, _C), jnp.int32),
             pltpu.VMEM((_CPS, _C), jnp.int32)]
            + [pltpu.VMEM((_C, _D), jnp.float32) for _ in range(_P)]
            + [pltpu.VMEM_SHARED((_NP, _D), jnp.float32)]
            + [pltpu.SemaphoreType.DMA for _ in range(2 * _P + 1)]
        ),
    )
    def _sc_aggregate(y_hbm, gidx_hbm, dst_hbm, zero_hbm, out_hbm,
                      gidx_v, dst_v, *rest):
        bufs = rest[:_P]
        acc_sh = rest[_P]
        gsems = rest[_P + 1:2 * _P + 1]
        ssems = rest[2 * _P + 1:3 * _P + 1]
        zsem = rest[3 * _P + 1]
        cid = lax.axis_index("c")
        sid = lax.axis_index("s")
        tid = cid * 16 + sid

        zdma = pltpu.async_copy(
            zero_hbm, acc_sh.at[pl.ds(sid * _RPT, _RPT)], zsem)
        slab_g = pltpu.async_copy(gidx_hbm.at[tid, 0], gidx_v, gsems[0])
        slab_d = pltpu.async_copy(dst_hbm.at[tid, 0], dst_v, gsems[1])
        slab_g.wait()
        slab_d.wait()
        zdma.wait()
        plsc.subcore_barrier()

        def seg_body(seg, carry):
            def gather(j):
                p = j % _P
                return pltpu.async_copy(
                    y_hbm.at[gidx_v.at[j]], bufs[p], gsems[p])

            copies = [None] * _CPS
            scats = [None] * _CPS
            for j in range(min(_P, _CPS)):
                copies[j] = gather(j)
            for j in range(_CPS):
                p = j % _P
                copies[j].wait()
                scats[j] = pltpu.async_copy(
                    bufs[p], acc_sh.at[dst_v.at[j]], ssems[p], add=True)
                k = j + _P - 1
                if _P <= k < _CPS:
                    scats[k - _P].wait()
                    copies[k] = gather(k)

            @pl.when(seg < _NSEG - 1)
            def _issue_gidx_slab():
                pltpu.async_copy(gidx_hbm.at[tid, seg + 1], gidx_v, gsems[0])

            for j in range(_CPS - _P, _CPS):
                scats[j].wait()

            @pl.when(seg < _NSEG - 1)
            def _next_slabs():
                pltpu.async_copy(dst_hbm.at[tid, seg + 1], dst_v,
                                 gsems[1]).wait()
                pltpu.make_async_copy(gidx_hbm.at[tid, 0], gidx_v,
                                      gsems[0]).wait()

            return carry

        lax.fori_loop(0, _NSEG, seg_body, 0)
        plsc.subcore_barrier()

        pltpu.sync_copy(acc_sh.at[pl.ds(sid * _RPT, _RPT)],
                        out_hbm.at[cid, pl.ds(sid * _RPT, _RPT)])

    return _sc_aggregate


def kernel(node_feat, edge_index, edge_feat, W_edge, b_edge,
           W_ih, W_hh, b_ih, b_hh):
    src = edge_index[0].astype(jnp.int32)
    dst = edge_index[1].astype(jnp.int32)

    ef_t = jnp.transpose(edge_feat).reshape(_NT, _E // _D, _D)
    src2 = src.reshape(_E // _D, _D)
    dst4 = dst.reshape(_NTILES, _NSEG, _CPS, _C)

    wt = jnp.transpose(W_edge, (0, 2, 1))
    b3 = b_edge.reshape(_NT, 1, _D)
    wiT = jnp.transpose(W_ih)
    whT = jnp.transpose(W_hh)
    bi2 = b_ih.reshape(1, 3 * _D)
    bh2 = b_hh.reshape(1, 3 * _D)
    zeros = jnp.zeros((_RPT, _D), jnp.float32)

    h = node_feat
    y4, gidx2 = _y0_call(h, wt, b3, ef_t, src2)
    y = y4.reshape(_NT * _N, _D)
    gidx4 = gidx2.reshape(_NTILES, _NSEG, _CPS, _C)
    for step in range(_STEPS):
        parts = _make_sc_aggregate()(y, gidx4, dst4, zeros)
        if step < _STEPS - 1:
            h, y4 = _gruy_call(parts, h, wiT, whT, bi2, bh2, wt, b3)
            y = y4.reshape(_NT * _N, _D)
        else:
            h = _gru_call(parts, h, wiT, whT, bi2, bh2)
    return h


# speedup vs baseline: 1.2598x; 1.0042x over previous
import functools

import jax
import jax.numpy as jnp
from jax import lax
from jax.experimental import pallas as pl
from jax.experimental.pallas import tpu as pltpu
from jax.experimental.pallas import tpu_sc as plsc

_N = 10000
_E = 320000
_D = 128
_NT = 4
_STEPS = 4

_NTILES = 32
_EPT = _E // _NTILES
_C = 80
_NCH = _EPT // _C
_NSEG = 5
_CPS = _NCH // _NSEG
_P = 3
_NP = 10240
_RPT = _NP // 16


def _gidx_body(ef_ref, src_ref, o_ref):
    best = ef_ref[0]
    idx = jnp.zeros_like(src_ref[...])
    for i in range(1, _NT):
        c = ef_ref[i]
        m = c > best
        idx = jnp.where(m, i, idx)
        best = jnp.where(m, c, best)
    o_ref[...] = idx * _N + src_ref[...]


def _gidx_call(ef_t, src2, interpret=False):
    return pl.pallas_call(
        _gidx_body,
        out_shape=jax.ShapeDtypeStruct((_E // _D, _D), jnp.int32),
        interpret=interpret,
    )(ef_t, src2)


def _y_body(h_ref, w_ref, b_ref, y_ref):
    y_ref[0] = (
        jnp.dot(h_ref[...], w_ref[0], preferred_element_type=jnp.float32)
        + b_ref[0]
    )


def _y_call(h, wt, b3, interpret=False):
    return pl.pallas_call(
        _y_body,
        grid=(_NT,),
        in_specs=[
            pl.BlockSpec((_N, _D), lambda i: (0, 0)),
            pl.BlockSpec((1, _D, _D), lambda i: (i, 0, 0)),
            pl.BlockSpec((1, 1, _D), lambda i: (i, 0, 0)),
        ],
        out_specs=pl.BlockSpec((1, _N, _D), lambda i: (i, 0, 0)),
        out_shape=jax.ShapeDtypeStruct((_NT, _N, _D), jnp.float32),
        interpret=interpret,
    )(h, wt, b3)


def _y0_body(h_ref, w_ref, b_ref, ef_ref, src_ref, y_ref, g_ref):
    y_ref[0] = (
        jnp.dot(h_ref[...], w_ref[0], preferred_element_type=jnp.float32)
        + b_ref[0]
    )

    @pl.when(pl.program_id(0) == 0)
    def _gidx():
        best = ef_ref[0]
        idx = jnp.zeros_like(src_ref[...])
        for i in range(1, _NT):
            c = ef_ref[i]
            m = c > best
            idx = jnp.where(m, i, idx)
            best = jnp.where(m, c, best)
        g_ref[...] = idx * _N + src_ref[...]


def _y0_call(h, wt, b3, ef_t, src2, interpret=False):
    return pl.pallas_call(
        _y0_body,
        grid=(_NT,),
        in_specs=[
            pl.BlockSpec((_N, _D), lambda i: (0, 0)),
            pl.BlockSpec((1, _D, _D), lambda i: (i, 0, 0)),
            pl.BlockSpec((1, 1, _D), lambda i: (i, 0, 0)),
            pl.BlockSpec((_NT, _E // _D, _D), lambda i: (0, 0, 0)),
            pl.BlockSpec((_E // _D, _D), lambda i: (0, 0)),
        ],
        out_specs=[
            pl.BlockSpec((1, _N, _D), lambda i: (i, 0, 0)),
            pl.BlockSpec((_E // _D, _D), lambda i: (0, 0)),
        ],
        out_shape=[
            jax.ShapeDtypeStruct((_NT, _N, _D), jnp.float32),
            jax.ShapeDtypeStruct((_E // _D, _D), jnp.int32),
        ],
        interpret=interpret,
    )(h, wt, b3, ef_t, src2)


_BN = 2000


def _gru_body(p_ref, h_ref, wi_ref, wh_ref, bi_ref, bh_ref, o_ref):
    agg = p_ref[0] + p_ref[1]
    h = h_ref[...]
    gi = jnp.dot(agg, wi_ref[...], preferred_element_type=jnp.float32) + bi_ref[...]
    gh = jnp.dot(h, wh_ref[...], preferred_element_type=jnp.float32) + bh_ref[...]
    r = jax.nn.sigmoid(gi[:, :_D] + gh[:, :_D])
    z = jax.nn.sigmoid(gi[:, _D:2 * _D] + gh[:, _D:2 * _D])
    n = jnp.tanh(gi[:, 2 * _D:] + r * gh[:, 2 * _D:])
    o_ref[...] = (1.0 - z) * n + z * h


def _gru_call(parts, h, wiT, whT, bi2, bh2, interpret=False):
    return pl.pallas_call(
        _gru_body,
        grid=(_N // _BN,),
        in_specs=[
            pl.BlockSpec((2, _BN, _D), lambda i: (0, i, 0)),
            pl.BlockSpec((_BN, _D), lambda i: (i, 0)),
            pl.BlockSpec((_D, 3 * _D), lambda i: (0, 0)),
            pl.BlockSpec((_D, 3 * _D), lambda i: (0, 0)),
            pl.BlockSpec((1, 3 * _D), lambda i: (0, 0)),
            pl.BlockSpec((1, 3 * _D), lambda i: (0, 0)),
        ],
        out_specs=pl.BlockSpec((_BN, _D), lambda i: (i, 0)),
        out_shape=jax.ShapeDtypeStruct((_N, _D), jnp.float32),
        interpret=interpret,
    )(parts, h, wiT, whT, bi2, bh2)


def _gruy_body(p_ref, h_ref, wi_ref, wh_ref, bi_ref, bh_ref, wt_ref, be_ref,
               o_ref, y_ref):
    agg = p_ref[0] + p_ref[1]
    h = h_ref[...]
    gi = jnp.dot(agg, wi_ref[...], preferred_element_type=jnp.float32) + bi_ref[...]
    gh = jnp.dot(h, wh_ref[...], preferred_element_type=jnp.float32) + bh_ref[...]
    r = jax.nn.sigmoid(gi[:, :_D] + gh[:, :_D])
    z = jax.nn.sigmoid(gi[:, _D:2 * _D] + gh[:, _D:2 * _D])
    n = jnp.tanh(gi[:, 2 * _D:] + r * gh[:, 2 * _D:])
    hn = (1.0 - z) * n + z * h
    o_ref[...] = hn
    for t in range(_NT):
        y_ref[t] = (
            jnp.dot(hn, wt_ref[t], preferred_element_type=jnp.float32)
            + be_ref[t]
        )


def _gruy_call(parts, h, wiT, whT, bi2, bh2, wt, b3, interpret=False):
    return pl.pallas_call(
        _gruy_body,
        grid=(_N // _BN,),
        in_specs=[
            pl.BlockSpec((2, _BN, _D), lambda i: (0, i, 0)),
            pl.BlockSpec((_BN, _D), lambda i: (i, 0)),
            pl.BlockSpec((_D, 3 * _D), lambda i: (0, 0)),
            pl.BlockSpec((_D, 3 * _D), lambda i: (0, 0)),
            pl.BlockSpec((1, 3 * _D), lambda i: (0, 0)),
            pl.BlockSpec((1, 3 * _D), lambda i: (0, 0)),
            pl.BlockSpec((_NT, _D, _D), lambda i: (0, 0, 0)),
            pl.BlockSpec((_NT, 1, _D), lambda i: (0, 0, 0)),
        ],
        out_specs=[
            pl.BlockSpec((_BN, _D), lambda i: (i, 0)),
            pl.BlockSpec((_NT, _BN, _D), lambda i: (0, i, 0)),
        ],
        out_shape=[
            jax.ShapeDtypeStruct((_N, _D), jnp.float32),
            jax.ShapeDtypeStruct((_NT, _N, _D), jnp.float32),
        ],
        interpret=interpret,
    )(parts, h, wiT, whT, bi2, bh2, wt, b3)


@functools.lru_cache(maxsize=1)
def _make_sc_aggregate():
    mesh = plsc.VectorSubcoreMesh(core_axis_name="c", subcore_axis_name="s")

    @functools.partial(
        pl.kernel,
        out_type=jax.ShapeDtypeStruct((2, _NP, _D), jnp.float32),
        mesh=mesh,
        scratch_types=(
            [pltpu.VMEM((_CPS, _C), jnp.int32),
             pltpu.VMEM((_CPS, _C), jnp.int32)]
            + [pltpu.VMEM((_C, _D), jnp.float32) for _ in range(_P)]
            + [pltpu.VMEM_SHARED((_NP, _D), jnp.float32)]
            + [pltpu.SemaphoreType.DMA for _ in range(2 * _P + 1)]
        ),
    )
    def _sc_aggregate(y_hbm, gidx_hbm, dst_hbm, zero_hbm, out_hbm,
                      gidx_v, dst_v, *rest):
        bufs = rest[:_P]
        acc_sh = rest[_P]
        gsems = rest[_P + 1:2 * _P + 1]
        ssems = rest[2 * _P + 1:3 * _P + 1]
        zsem = rest[3 * _P + 1]
        cid = lax.axis_index("c")
        sid = lax.axis_index("s")
        tid = cid * 16 + sid

        zdma = pltpu.async_copy(
            zero_hbm, acc_sh.at[pl.ds(sid * _RPT, _RPT)], zsem)
        slab_g = pltpu.async_copy(gidx_hbm.at[tid, 0], gidx_v, gsems[0])
        slab_d = pltpu.async_copy(dst_hbm.at[tid, 0], dst_v, gsems[1])
        slab_g.wait()
        slab_d.wait()

        def _gather(j, p):
            return pltpu.async_copy(
                y_hbm.at[gidx_v.at[j]], bufs[p], gsems[p])

        for j in range(_P):
            _gather(j, j)
        zdma.wait()
        plsc.subcore_barrier()

        def seg_body(seg, carry):
            def gather(j):
                return _gather(j, j % _P)

            copies = [None] * _CPS
            scats = [None] * _CPS

            @pl.when(seg > 0)
            def _prime():
                for j in range(min(_P, _CPS)):
                    gather(j)

            for j in range(min(_P, _CPS)):
                copies[j] = pltpu.make_async_copy(
                    y_hbm.at[gidx_v.at[j]], bufs[j % _P], gsems[j % _P])
            for j in range(_CPS):
                p = j % _P
                copies[j].wait()
                scats[j] = pltpu.async_copy(
                    bufs[p], acc_sh.at[dst_v.at[j]], ssems[p], add=True)
                k = j + _P - 1
                if _P <= k < _CPS:
                    scats[k - _P].wait()
                    copies[k] = gather(k)

            @pl.when(seg < _NSEG - 1)
            def _issue_gidx_slab():
                pltpu.async_copy(gidx_hbm.at[tid, seg + 1], gidx_v, gsems[0])

            for j in range(_CPS - _P, _CPS):
                scats[j].wait()

            @pl.when(seg < _NSEG - 1)
            def _next_slabs():
                pltpu.async_copy(dst_hbm.at[tid, seg + 1], dst_v,
                                 gsems[1]).wait()
                pltpu.make_async_copy(gidx_hbm.at[tid, 0], gidx_v,
                                      gsems[0]).wait()

            return carry

        lax.fori_loop(0, _NSEG, seg_body, 0)
        plsc.subcore_barrier()

        pltpu.sync_copy(acc_sh.at[pl.ds(sid * _RPT, _RPT)],
                        out_hbm.at[cid, pl.ds(sid * _RPT, _RPT)])

    return _sc_aggregate


def kernel(node_feat, edge_index, edge_feat, W_edge, b_edge,
           W_ih, W_hh, b_ih, b_hh):
    src = edge_index[0].astype(jnp.int32)
    dst = edge_index[1].astype(jnp.int32)

    ef_t = jnp.transpose(edge_feat).reshape(_NT, _E // _D, _D)
    src2 = src.reshape(_E // _D, _D)
    dst4 = dst.reshape(_NTILES, _NSEG, _CPS, _C)

    wt = jnp.transpose(W_edge, (0, 2, 1))
    b3 = b_edge.reshape(_NT, 1, _D)
    wiT = jnp.transpose(W_ih)
    whT = jnp.transpose(W_hh)
    bi2 = b_ih.reshape(1, 3 * _D)
    bh2 = b_hh.reshape(1, 3 * _D)
    zeros = jnp.zeros((_RPT, _D), jnp.float32)

    h = node_feat
    y4, gidx2 = _y0_call(h, wt, b3, ef_t, src2)
    y = y4.reshape(_NT * _N, _D)
    gidx4 = gidx2.reshape(_NTILES, _NSEG, _CPS, _C)
    for step in range(_STEPS):
        parts = _make_sc_aggregate()(y, gidx4, dst4, zeros)
        if step < _STEPS - 1:
            h, y4 = _gruy_call(parts, h, wiT, whT, bi2, bh2, wt, b3)
            y = y4.reshape(_NT * _N, _D)
        else:
            h = _gru_call(parts, h, wiT, whT, bi2, bh2)
    return h
